# baseline (speedup 1.0000x reference)
import functools

import jax
import jax.numpy as jnp
from jax import lax
from jax.experimental import pallas as pl
from jax.experimental.pallas import tpu as pltpu
from jax.experimental.pallas import tpu_sc as plsc

_E = 64
_K = 8
_T = 512
_N = 8192
_NC = 2
_NS = 16
_C = _N // (_NC * _NS)
_G = 16


def _logits_body(x_ref, w_ref, b_ref, lt_ref):
    logits = jnp.dot(x_ref[...], w_ref[...],
                     preferred_element_type=jnp.float32) + b_ref[...]
    lt_ref[...] = logits.T


def _tree(op, vals):
    vals = list(vals)
    while len(vals) > 1:
        nxt = [op(vals[i], vals[i + 1]) for i in range(0, len(vals) - 1, 2)]
        if len(vals) % 2:
            nxt.append(vals[-1])
        vals = nxt
    return vals[0]


_sc_mesh = plsc.VectorSubcoreMesh(core_axis_name="c", subcore_axis_name="s")


@functools.partial(
    pl.kernel,
    mesh=_sc_mesh,
    out_type=[
        jax.ShapeDtypeStruct((_E, _N), jnp.float32),
        jax.ShapeDtypeStruct((_E, _N), jnp.float32),
    ],
    scratch_types=[
        pltpu.VMEM((_E, _C), jnp.float32),
        pltpu.VMEM((_E, _C), jnp.float32),
        pltpu.VMEM((_E, _C), jnp.float32),
    ],
)
def _sc_tail(lt_hbm, w_hbm, g_hbm, ltile, wtile, gtile):
    wid = lax.axis_index("s") * _NC + lax.axis_index("c")
    base = wid * _C
    pltpu.sync_copy(lt_hbm.at[:, pl.ds(base, _C)], ltile)

    def group(g, carry):
        sl = pl.ds(g * _G, _G)
        l = [ltile[e, sl] for e in range(_E)]
        m = _tree(jnp.maximum, l)
        ev = [jnp.exp(v - m) for v in l]
        s = _tree(jnp.add, ev)
        for e in range(_E):
            wtile[e, sl] = ev[e] / s
        wv = [ev[e] / s for e in range(_E)]
        for _ in range(_K):
            cur = _tree(jnp.maximum, wv)
            wv = [jnp.where(v == cur, -1.0, v) for v in wv]
        for e in range(_E):
            gtile[e, sl] = jnp.where(wv[e] < 0.0, wtile[e, sl], 0.0)
        return carry

    lax.fori_loop(0, _C // _G, group, 0)
    pltpu.sync_copy(wtile, w_hbm.at[:, pl.ds(base, _C)])
    pltpu.sync_copy(gtile, g_hbm.at[:, pl.ds(base, _C)])


def kernel(x, W, b):
    B, S, D = x.shape
    N = B * S
    x2 = x.reshape(N, D)
    b2 = b.reshape(1, _E)
    lt = pl.pallas_call(
        _logits_body,
        grid=(N // _T,),
        in_specs=[
            pl.BlockSpec((_T, D), lambda i: (i, 0)),
            pl.BlockSpec((D, _E), lambda i: (0, 0)),
            pl.BlockSpec((1, _E), lambda i: (0, 0)),
        ],
        out_specs=pl.BlockSpec((_E, _T), lambda i: (0, i)),
        out_shape=jax.ShapeDtypeStruct((_E, N), jnp.float32),
    )(x2, W, b2)
    weights, gated = _sc_tail(lt)
    return gated.reshape(_E, B, S), weights.reshape(_E, B, S)


# speedup vs baseline: 1.5310x; 1.5310x over previous
import jax
import jax.numpy as jnp
from jax.experimental import pallas as pl

_E = 64
_K = 8
_T = 512


def _gating_body(x_ref, w_ref, b_ref, gated_ref, weights_ref):
    x = x_ref[...]
    w = w_ref[...]
    b = b_ref[...]
    logits = jnp.dot(x, w, preferred_element_type=jnp.float32) + b
    lt = logits.T
    m = jnp.max(lt, axis=0, keepdims=True)
    e = jnp.exp(lt - m)
    s = jnp.sum(e, axis=0, keepdims=True)
    weights = e / s

    wv = weights
    for _ in range(_K):
        cur = jnp.max(wv, axis=0, keepdims=True)
        wv = jnp.where(wv == cur, -1.0, wv)
    gated_ref[...] = jnp.where(wv < 0, weights, 0.0)
    weights_ref[...] = weights


def kernel(x, W, b):
    B, S, D = x.shape
    N = B * S
    x2 = x.reshape(N, D)
    b2 = b.reshape(1, _E)
    grid = (N // _T,)
    out_shape = [
        jax.ShapeDtypeStruct((_E, N), jnp.float32),
        jax.ShapeDtypeStruct((_E, N), jnp.float32),
    ]
    gated, weights = pl.pallas_call(
        _gating_body,
        grid=grid,
        in_specs=[
            pl.BlockSpec((_T, D), lambda i: (i, 0)),
            pl.BlockSpec((D, _E), lambda i: (0, 0)),
            pl.BlockSpec((1, _E), lambda i: (0, 0)),
        ],
        out_specs=[
            pl.BlockSpec((_E, _T), lambda i: (0, i)),
            pl.BlockSpec((_E, _T), lambda i: (0, i)),
        ],
        out_shape=out_shape,
    )(x2, W, b2)
    return gated.reshape(_E, B, S), weights.reshape(_E, B, S)
